# Initial kernel scaffold; baseline (speedup 1.0000x reference)
import jax
import jax.numpy as jnp
from jax.experimental import pallas as pl


def kernel(feat, edge_index, w, W_f_in, b_f_in, W_t_in, b_t_in, W_t_out, b_t_out, W_f_out, b_f_out):
    raise NotImplementedError("write your pallas kernel here")


# speedup vs baseline: 4.3166x; 4.3166x over previous
import dataclasses
import functools

import jax
import jax.numpy as jnp
from jax import lax
from jax.experimental import pallas as pl
from jax.experimental.pallas import tpu as pltpu
from jax.experimental.pallas import tpu_sc as plsc

_N = 10000
_E = 320000
_IWIN = 12
_OWIN = 12
_HWIN = 4
_IDIM = 128
_HDIM = 16

_BN = 1000
_NB = _N // _BN

_C = 128
_NCHUNK = _E // _C
_NW = 32
_ITERS = -(-_NCHUNK // _NW)
_SEG = 80
_NSEG = _N // _SEG


def _front_body(feat_ref, wf_ref, bf_ref, wt_ref, bt_ref, out_ref):
    acc = [None] * _HWIN
    for t in range(_IWIN):
        xt = feat_ref[:, t, :]
        x1 = jnp.dot(xt, wf_ref[...], preferred_element_type=jnp.float32)
        x1 = jnp.maximum(x1 + bf_ref[...], 0.0)
        for k in range(_HWIN):
            term = x1 * wt_ref[t, k]
            acc[k] = term if t == 0 else acc[k] + term
    for k in range(_HWIN):
        sl = slice(k * _HDIM, (k + 1) * _HDIM)
        out_ref[:, sl] = jnp.maximum(acc[k] + bt_ref[0, k], 0.0)


def _front(feat, wf, bf, wt, bt):
    return pl.pallas_call(
        _front_body,
        grid=(_NB,),
        in_specs=[
            pl.BlockSpec((_BN, _IWIN, _IDIM), lambda i: (i, 0, 0)),
            pl.BlockSpec((_IDIM, _HDIM), lambda i: (0, 0)),
            pl.BlockSpec((1, _HDIM), lambda i: (0, 0)),
            pl.BlockSpec(memory_space=pltpu.SMEM),
            pl.BlockSpec(memory_space=pltpu.SMEM),
        ],
        out_specs=pl.BlockSpec((_BN, _HWIN * _HDIM), lambda i: (i, 0)),
        out_shape=jax.ShapeDtypeStruct((_N, _HWIN * _HDIM), jnp.float32),
    )(feat, wf, bf, wt, bt)


_MESH = plsc.VectorSubcoreMesh(core_axis_name="c", subcore_axis_name="s")

_SC_PARAMS = pltpu.CompilerParams()
for _f, _v in (("needs_layout_passes", False), ("use_tc_tiling_on_sc", False)):
    if _f in pltpu.CompilerParams.__dataclass_fields__:
        _SC_PARAMS = dataclasses.replace(_SC_PARAMS, **{_f: _v})


@functools.partial(
    pl.kernel,
    mesh=_MESH,
    compiler_params=_SC_PARAMS,
    out_type=[
        jax.ShapeDtypeStruct((2, _N, _HWIN * _HDIM), jnp.float32),
        jax.ShapeDtypeStruct((2, _N, 16), jnp.float32),
    ],
    scratch_types=[
        pltpu.VMEM_SHARED((_N, _HWIN * _HDIM), jnp.float32),
        pltpu.VMEM_SHARED((_N, 16), jnp.float32),
        pltpu.VMEM((_SEG, _HWIN * _HDIM), jnp.float32),
        pltpu.VMEM((_SEG, 16), jnp.float32),
        pltpu.VMEM((_C, _HWIN * _HDIM), jnp.float32),
        pltpu.VMEM((_C, 16), jnp.float32),
        pltpu.VMEM((_C,), jnp.int32),
        pltpu.VMEM((_C,), jnp.int32),
        pltpu.VMEM((_C,), jnp.float32),
        pltpu.SemaphoreType.DMA,
    ],
)
def _sc_graph(x2_hbm, src_hbm, dst_hbm, w_hbm, h_out, d_out,
              h_acc, d_acc, zh, zd, rows, ones, sidx, didx, wsm, sem):
    c = lax.axis_index("c")
    s = lax.axis_index("s")
    wid = s * 2 + c

    @pl.loop(0, _SEG)
    def _(r):
        for k in range(_HWIN):
            zh[r, pl.ds(k * 16, 16)] = jnp.zeros((16,), jnp.float32)
        zd[r, pl.ds(0, 16)] = jnp.zeros((16,), jnp.float32)

    @pl.loop(0, _C)
    def _(r):
        ones[r, pl.ds(0, 16)] = jnp.ones((16,), jnp.float32)

    @pl.loop(0, 8)
    def _(j):
        seg = s + 16 * j

        @pl.when(seg < _NSEG)
        def _():
            pltpu.sync_copy(zh, h_acc.at[pl.ds(seg * _SEG, _SEG)])
            pltpu.sync_copy(zd, d_acc.at[pl.ds(seg * _SEG, _SEG)])

    plsc.subcore_barrier()

    @pl.loop(0, _ITERS)
    def _(t):
        chunk = wid + _NW * t

        @pl.when(chunk < _NCHUNK)
        def _():
            off = chunk * _C
            pltpu.sync_copy(src_hbm.at[pl.ds(off, _C)], sidx)
            pltpu.sync_copy(dst_hbm.at[pl.ds(off, _C)], didx)
            pltpu.sync_copy(w_hbm.at[pl.ds(off, _C)], wsm)
            pltpu.async_copy(x2_hbm.at[sidx], rows, sem).wait()

            @pl.loop(0, _C)
            def _(e):
                ws = plsc.load_gather(wsm, [jnp.full((16,), e, jnp.int32)])
                for k in range(_HWIN):
                    rows[e, pl.ds(k * 16, 16)] = rows[e, pl.ds(k * 16, 16)] * ws

            pltpu.sync_copy(rows, h_acc.at[didx], add=True)
            pltpu.sync_copy(ones, d_acc.at[didx], add=True)

    plsc.subcore_barrier()

    @pl.loop(0, 8)
    def _(j):
        seg = s + 16 * j

        @pl.when(seg < _NSEG)
        def _():
            off = seg * _SEG
            pltpu.sync_copy(h_acc.at[pl.ds(off, _SEG)],
                            h_out.at[c, pl.ds(off, _SEG)])
            pltpu.sync_copy(d_acc.at[pl.ds(off, _SEG)],
                            d_out.at[c, pl.ds(off, _SEG)])


def _back_body(hp_ref, dp_ref, wto_ref, bto_ref, wfo_ref, bfo_ref, out_ref):
    d = dp_ref[0] + dp_ref[1]
    r = 1.0 / jnp.maximum(d, 1.0)
    x3 = []
    for k in range(_HWIN):
        sl = slice(k * _HDIM, (k + 1) * _HDIM)
        x3.append(jnp.maximum((hp_ref[0, :, sl] + hp_ref[1, :, sl]) * r, 0.0))
    cols = []
    for t in range(_OWIN):
        y = x3[0] * wto_ref[0, t]
        for k in range(1, _HWIN):
            y = y + x3[k] * wto_ref[k, t]
        y = jnp.maximum(y + bto_ref[0, t], 0.0)
        cols.append(jnp.sum(y * wfo_ref[...], axis=1, keepdims=True)
                    + bfo_ref[0, 0])
    out_ref[...] = jnp.concatenate(cols, axis=1)


def _back(hp, dp, wto, bto, wfo, bfo):
    return pl.pallas_call(
        _back_body,
        grid=(_NB,),
        in_specs=[
            pl.BlockSpec((2, _BN, _HWIN * _HDIM), lambda i: (0, i, 0)),
            pl.BlockSpec((2, _BN, 16), lambda i: (0, i, 0)),
            pl.BlockSpec(memory_space=pltpu.SMEM),
            pl.BlockSpec(memory_space=pltpu.SMEM),
            pl.BlockSpec((1, _HDIM), lambda i: (0, 0)),
            pl.BlockSpec(memory_space=pltpu.SMEM),
        ],
        out_specs=pl.BlockSpec((_BN, _OWIN), lambda i: (i, 0)),
        out_shape=jax.ShapeDtypeStruct((_N, _OWIN), jnp.float32),
    )(hp, dp, wto, bto, wfo, bfo)


def kernel(feat, edge_index, w, W_f_in, b_f_in, W_t_in, b_t_in,
           W_t_out, b_t_out, W_f_out, b_f_out):
    featsq = feat.reshape(_N, _IWIN, _IDIM)
    x2 = _front(featsq, W_f_in, b_f_in.reshape(1, _HDIM),
                W_t_in, b_t_in.reshape(1, _HWIN))
    src = edge_index[0]
    dst = edge_index[1]
    h2, d2 = _sc_graph(x2, src, dst, w)
    out = _back(h2, d2, W_t_out, b_t_out.reshape(1, _OWIN),
                W_f_out.reshape(1, _HDIM), b_f_out.reshape(1, 1))
    return out.reshape(1, _N, _OWIN, 1)


# speedup vs baseline: 5.9567x; 1.3799x over previous
import dataclasses
import functools

import jax
import jax.numpy as jnp
from jax import lax
from jax.experimental import pallas as pl
from jax.experimental.pallas import tpu as pltpu
from jax.experimental.pallas import tpu_sc as plsc

_N = 10000
_E = 320000
_IWIN = 12
_OWIN = 12
_HWIN = 4
_IDIM = 128
_HDIM = 16

_BN = 1000
_NB = _N // _BN

_C = 128
_NCHUNK = _E // _C
_NW = 32
_OWN = _NCHUNK // _NW
_BATCH = 6
_NBATCH = _OWN // _BATCH
_REM = _NCHUNK - _OWN * _NW
_SEG = 80
_NSEG = _N // _SEG


def _front_body(feat_ref, wf_ref, bf_ref, wt_ref, bt_ref, out_ref):
    acc = [None] * _HWIN
    for t in range(_IWIN):
        xt = feat_ref[:, t, :]
        x1 = jnp.dot(xt, wf_ref[...], preferred_element_type=jnp.float32)
        x1 = jnp.maximum(x1 + bf_ref[...], 0.0)
        for k in range(_HWIN):
            term = x1 * wt_ref[t, k]
            acc[k] = term if t == 0 else acc[k] + term
    for k in range(_HWIN):
        sl = slice(k * _HDIM, (k + 1) * _HDIM)
        out_ref[:, sl] = jnp.maximum(acc[k] + bt_ref[0, k], 0.0)


def _front(feat, wf, bf, wt, bt):
    return pl.pallas_call(
        _front_body,
        grid=(_NB,),
        in_specs=[
            pl.BlockSpec((_BN, _IWIN, _IDIM), lambda i: (i, 0, 0)),
            pl.BlockSpec((_IDIM, _HDIM), lambda i: (0, 0)),
            pl.BlockSpec((1, _HDIM), lambda i: (0, 0)),
            pl.BlockSpec(memory_space=pltpu.SMEM),
            pl.BlockSpec(memory_space=pltpu.SMEM),
        ],
        out_specs=pl.BlockSpec((_BN, _HWIN * _HDIM), lambda i: (i, 0)),
        out_shape=jax.ShapeDtypeStruct((_N, _HWIN * _HDIM), jnp.float32),
    )(feat, wf, bf, wt, bt)


_MESH = plsc.VectorSubcoreMesh(core_axis_name="c", subcore_axis_name="s")

_SC_PARAMS = pltpu.CompilerParams()
for _f, _v in (("needs_layout_passes", False), ("use_tc_tiling_on_sc", False)):
    if _f in pltpu.CompilerParams.__dataclass_fields__:
        _SC_PARAMS = dataclasses.replace(_SC_PARAMS, **{_f: _v})


@functools.partial(
    pl.kernel,
    mesh=_MESH,
    compiler_params=_SC_PARAMS,
    out_type=[
        jax.ShapeDtypeStruct((2, _N, _HWIN * _HDIM), jnp.float32),
        jax.ShapeDtypeStruct((2, _N, 16), jnp.float32),
    ],
    scratch_types=[
        pltpu.VMEM_SHARED((_N, _HWIN * _HDIM), jnp.float32),
        pltpu.VMEM_SHARED((_N, 16), jnp.float32),
        pltpu.VMEM((_C, _HWIN * _HDIM), jnp.float32),
        pltpu.VMEM((_C, _HWIN * _HDIM), jnp.float32),
        pltpu.VMEM((_C, 16), jnp.float32),
        pltpu.VMEM((_BATCH * _C,), jnp.int32),
        pltpu.VMEM((_BATCH * _C,), jnp.int32),
        pltpu.VMEM((_BATCH * _C,), jnp.float32),
        pltpu.SemaphoreType.DMA,
        pltpu.SemaphoreType.DMA,
    ],
)
def _sc_graph(x2_hbm, src_hbm, dst_hbm, w_hbm, h_out, d_out,
              h_acc, d_acc, rows0, rows1, ones, sidx, didx, wsm, sem0, sem1):
    c = lax.axis_index("c")
    s = lax.axis_index("s")
    wid = s * 2 + c

    @pl.loop(0, _C)
    def _(r):
        for k in range(_HWIN):
            rows0[r, pl.ds(k * 16, 16)] = jnp.zeros((16,), jnp.float32)
        ones[r, pl.ds(0, 16)] = jnp.zeros((16,), jnp.float32)

    @pl.loop(0, 8)
    def _(j):
        seg = s + 16 * j

        @pl.when(seg < _NSEG)
        def _():
            pltpu.sync_copy(rows0.at[pl.ds(0, _SEG)],
                            h_acc.at[pl.ds(seg * _SEG, _SEG)])
            pltpu.sync_copy(ones.at[pl.ds(0, _SEG)],
                            d_acc.at[pl.ds(seg * _SEG, _SEG)])

    @pl.loop(0, _C)
    def _(r):
        ones[r, pl.ds(0, 16)] = jnp.ones((16,), jnp.float32)

    plsc.subcore_barrier()

    sems = (sem0, sem1)
    bufs = (rows0, rows1)

    def _scale_and_scatter(buf, base_e):
        @pl.loop(0, _C)
        def _(e):
            idx = jnp.full((16,), base_e, jnp.int32) + e
            ws = plsc.load_gather(wsm, [idx])
            for k in range(_HWIN):
                buf[e, pl.ds(k * 16, 16)] = buf[e, pl.ds(k * 16, 16)] * ws

        dv = didx.at[pl.ds(base_e, _C)]
        pltpu.sync_copy(buf, h_acc.at[dv], add=True)
        pltpu.sync_copy(ones, d_acc.at[dv], add=True)

    @pl.loop(0, _NBATCH)
    def _(b):
        eoff = (wid * _OWN + b * _BATCH) * _C
        pltpu.sync_copy(src_hbm.at[pl.ds(eoff, _BATCH * _C)], sidx)
        pltpu.sync_copy(dst_hbm.at[pl.ds(eoff, _BATCH * _C)], didx)
        pltpu.sync_copy(w_hbm.at[pl.ds(eoff, _BATCH * _C)], wsm)

        cps = [None, None]
        cps[0] = pltpu.async_copy(
            x2_hbm.at[sidx.at[pl.ds(0, _C)]], rows0, sem0)
        for i in range(_BATCH):
            if i + 1 < _BATCH:
                cps[(i + 1) % 2] = pltpu.async_copy(
                    x2_hbm.at[sidx.at[pl.ds((i + 1) * _C, _C)]],
                    bufs[(i + 1) % 2], sems[(i + 1) % 2])
            cps[i % 2].wait()
            _scale_and_scatter(bufs[i % 2], i * _C)

    @pl.when(wid < _REM)
    def _():
        eoff = (_OWN * _NW + wid) * _C
        pltpu.sync_copy(src_hbm.at[pl.ds(eoff, _C)], sidx.at[pl.ds(0, _C)])
        pltpu.sync_copy(dst_hbm.at[pl.ds(eoff, _C)], didx.at[pl.ds(0, _C)])
        pltpu.sync_copy(w_hbm.at[pl.ds(eoff, _C)], wsm.at[pl.ds(0, _C)])
        pltpu.async_copy(x2_hbm.at[sidx.at[pl.ds(0, _C)]], rows0, sem0).wait()
        _scale_and_scatter(rows0, 0)

    plsc.subcore_barrier()

    @pl.loop(0, 8)
    def _(j):
        seg = s + 16 * j

        @pl.when(seg < _NSEG)
        def _():
            off = seg * _SEG
            pltpu.sync_copy(h_acc.at[pl.ds(off, _SEG)],
                            h_out.at[c, pl.ds(off, _SEG)])
            pltpu.sync_copy(d_acc.at[pl.ds(off, _SEG)],
                            d_out.at[c, pl.ds(off, _SEG)])


def _back_body(hp_ref, dp_ref, wto_ref, bto_ref, wfo_ref, bfo_ref, out_ref):
    d = dp_ref[0] + dp_ref[1]
    r = 1.0 / jnp.maximum(d, 1.0)
    x3 = []
    for k in range(_HWIN):
        sl = slice(k * _HDIM, (k + 1) * _HDIM)
        x3.append(jnp.maximum((hp_ref[0, :, sl] + hp_ref[1, :, sl]) * r, 0.0))
    cols = []
    for t in range(_OWIN):
        y = x3[0] * wto_ref[0, t]
        for k in range(1, _HWIN):
            y = y + x3[k] * wto_ref[k, t]
        y = jnp.maximum(y + bto_ref[0, t], 0.0)
        cols.append(jnp.sum(y * wfo_ref[...], axis=1, keepdims=True)
                    + bfo_ref[0, 0])
    out_ref[...] = jnp.concatenate(cols, axis=1)


def _back(hp, dp, wto, bto, wfo, bfo):
    return pl.pallas_call(
        _back_body,
        grid=(_NB,),
        in_specs=[
            pl.BlockSpec((2, _BN, _HWIN * _HDIM), lambda i: (0, i, 0)),
            pl.BlockSpec((2, _BN, 16), lambda i: (0, i, 0)),
            pl.BlockSpec(memory_space=pltpu.SMEM),
            pl.BlockSpec(memory_space=pltpu.SMEM),
            pl.BlockSpec((1, _HDIM), lambda i: (0, 0)),
            pl.BlockSpec(memory_space=pltpu.SMEM),
        ],
        out_specs=pl.BlockSpec((_BN, _OWIN), lambda i: (i, 0)),
        out_shape=jax.ShapeDtypeStruct((_N, _OWIN), jnp.float32),
    )(hp, dp, wto, bto, wfo, bfo)


def kernel(feat, edge_index, w, W_f_in, b_f_in, W_t_in, b_t_in,
           W_t_out, b_t_out, W_f_out, b_f_out):
    featsq = feat.reshape(_N, _IWIN, _IDIM)
    x2 = _front(featsq, W_f_in, b_f_in.reshape(1, _HDIM),
                W_t_in, b_t_in.reshape(1, _HWIN))
    src = edge_index[0]
    dst = edge_index[1]
    h2, d2 = _sc_graph(x2, src, dst, w)
    out = _back(h2, d2, W_t_out, b_t_out.reshape(1, _OWIN),
                W_f_out.reshape(1, _HDIM), b_f_out.reshape(1, 1))
    return out.reshape(1, _N, _OWIN, 1)
